# baseline (speedup 1.0000x reference)
import jax
import jax.numpy as jnp
from jax import lax
from jax.experimental import pallas as pl
from jax.experimental.pallas import tpu as pltpu
from jax.experimental.pallas import tpu_sc as plsc

NUM_EMBED = 1000000
EMBED_DIM = 32
SHARED_DIM = 8
BATCH = 16384

NC = 2
NS = 16
NW = NC * NS
B_PER_W = BATCH // NW
CHUNK = 128
NCHUNK = B_PER_W // CHUNK
LANES = 16


def _body(idx_hbm, sh_hbm, table_hbm, out_hbm, idx_v, rows_v, sem):
    wid = lax.axis_index("s") * NC + lax.axis_index("c")
    base = wid * B_PER_W

    pltpu.sync_copy(idx_hbm.at[pl.ds(wid * NCHUNK, NCHUNK)], idx_v)

    copies = []
    for j in range(NCHUNK):
        copies.append(
            pltpu.async_copy(
                table_hbm.at[idx_v.at[j]],
                rows_v.at[pl.ds(j * CHUNK, CHUNK)],
                sem,
            )
        )
    for c in copies:
        c.wait()

    pltpu.sync_copy(sh_hbm, rows_v.at[:, pl.ds(0, SHARED_DIM)])

    pltpu.sync_copy(rows_v, out_hbm.at[pl.ds(base, B_PER_W)])


@jax.jit
def kernel(X, table, shared_embed):
    idx = X.astype(jnp.int32).reshape(NW * NCHUNK, CHUNK)
    shblk = jnp.broadcast_to(shared_embed, (B_PER_W, SHARED_DIM))
    mesh = plsc.VectorSubcoreMesh(core_axis_name="c", subcore_axis_name="s")
    k = pl.kernel(
        _body,
        mesh=mesh,
        compiler_params=pltpu.CompilerParams(use_tc_tiling_on_sc=False),
        out_type=jax.ShapeDtypeStruct((BATCH, EMBED_DIM), jnp.float32),
        scratch_types=[
            pltpu.VMEM((NCHUNK, CHUNK), jnp.int32),
            pltpu.VMEM((B_PER_W, EMBED_DIM), jnp.float32),
            pltpu.SemaphoreType.DMA,
        ],
    )
    return k(idx, shblk, table)


# speedup vs baseline: 5.3315x; 5.3315x over previous
import jax
import jax.numpy as jnp
from jax import lax
from jax.experimental import pallas as pl
from jax.experimental.pallas import tpu as pltpu
from jax.experimental.pallas import tpu_sc as plsc

NUM_EMBED = 1000000
EMBED_DIM = 32
SHARED_DIM = 8
BATCH = 16384

NC = 2
NS = 16
NW = NC * NS
L = 16
GROWS = EMBED_DIM - SHARED_DIM
PIECE = 1024
NPF = NUM_EMBED // PIECE
TAIL512_LO = NPF * PIECE
TAIL64_LO = TAIL512_LO + 512
BITS_B = 14


def _body(x_hbm, sh_hbm, table_hbm, out_hbm,
          idx_v, pairs_v, ring0_v, ring1_v, tail512_v, tail64_v,
          stage_v, sh_v, sem0, sem1, semo):
    wid = lax.axis_index("s") * NC + lax.axis_index("c")
    lanes = lax.iota(jnp.int32, L)
    zeros = jnp.zeros((L,), jnp.int32)

    start_piece = jnp.where(wid < 16, 31 * wid, 30 * wid + 16)
    npieces = jnp.where(wid < 16, 31, 30)
    lo = start_piece * PIECE
    width = jnp.where(wid == 31, 30 * PIECE + 576, npieces * PIECE)

    pltpu.sync_copy(x_hbm, idx_v)
    pltpu.sync_copy(sh_hbm, sh_v)

    def fire(p, buf, sem):
        off = pl.multiple_of((start_piece + p) * PIECE, PIECE)
        return pltpu.async_copy(
            table_hbm.at[pl.ds(SHARED_DIM, GROWS), pl.ds(off, PIECE)],
            buf, sem)

    @pl.when(npieces > 0)
    def _():
        fire(0, ring0_v, sem0)

    def prebin(g, cnt):
        off = pl.multiple_of(g * L, L)
        v = idx_v[pl.ds(off, L)]
        li = v - lo
        m = jnp.logical_and(li >= 0, li < width)
        pos = cnt + plsc.cumsum(jnp.where(m, 1, 0)) - 1
        packed = li * BATCH + (g * L + lanes)
        plsc.store_scatter(pairs_v, [pos], packed, mask=m)
        return cnt + plsc.all_reduce_population_count(m)

    cnt = lax.fori_loop(0, BATCH // L, prebin, zeros)
    cnt_s = cnt[0]

    vsh = plsc.load_gather(sh_v, [zeros, jnp.bitwise_and(lanes, SHARED_DIM - 1)])
    for slot in range(L):
        stage_v[pl.ds(slot * EMBED_DIM, L)] = vsh

    def process(buf, plo, clamp):
        pwidth = buf.shape[1]

        def compact(g, pcnt):
            off = pl.multiple_of(g * L, L)
            pv = pairs_v[pl.ds(off, L)]
            li = lax.shift_right_logical(pv, BITS_B)
            b = jnp.bitwise_and(pv, BATCH - 1)
            ll = li - plo
            m = jnp.logical_and(g * L + lanes < cnt_s,
                                jnp.logical_and(ll >= 0, ll < pwidth))
            pos = pcnt + plsc.cumsum(jnp.where(m, 1, 0)) - 1
            plsc.store_scatter(idx_v, [pos], ll * BATCH + b, mask=m)
            return pcnt + plsc.all_reduce_population_count(m)

        pcnt = lax.fori_loop(0, (cnt_s + L - 1) // L, compact, zeros)
        pcnt_s = pcnt[0]

        def batch(t, carry):
            off = pl.multiple_of(t * L, L)
            bv = idx_v[pl.ds(off, L)]
            ll = jnp.bitwise_and(lax.shift_right_logical(bv, BITS_B), clamp)
            b = jnp.bitwise_and(bv, BATCH - 1)
            for j in range(GROWS):
                gj = plsc.load_gather(
                    buf, [jnp.full((L,), j, jnp.int32), ll])
                plsc.store_scatter(
                    stage_v, [lanes * EMBED_DIM + SHARED_DIM + j], gj)
            rem = pcnt_s - t * L
            for u in range(L):
                @pl.when(rem > u)
                def _():
                    bo = pl.multiple_of(b[u] * EMBED_DIM, EMBED_DIM)
                    pltpu.async_copy(
                        stage_v.at[pl.ds(u * EMBED_DIM, EMBED_DIM)],
                        out_hbm.at[pl.ds(bo, EMBED_DIM)], semo)
            for u in range(L):
                @pl.when(rem > u)
                def _():
                    pltpu.make_async_copy(
                        stage_v.at[pl.ds(0, EMBED_DIM)],
                        out_hbm.at[pl.ds(0, EMBED_DIM)], semo).wait()
            return carry

        lax.fori_loop(0, (pcnt_s + L - 1) // L, batch, 0)

    def piece_step(p, carry):
        @pl.when(p + 1 < npieces)
        def _():
            @pl.when(jnp.bitwise_and(p + 1, 1) == 0)
            def _():
                fire(p + 1, ring0_v, sem0)
            @pl.when(jnp.bitwise_and(p + 1, 1) == 1)
            def _():
                fire(p + 1, ring1_v, sem1)

        @pl.when(jnp.bitwise_and(p, 1) == 0)
        def _():
            pltpu.make_async_copy(
                table_hbm.at[pl.ds(SHARED_DIM, GROWS), pl.ds(0, PIECE)],
                ring0_v, sem0).wait()
            process(ring0_v, p * PIECE, PIECE - 1)

        @pl.when(jnp.bitwise_and(p, 1) == 1)
        def _():
            pltpu.make_async_copy(
                table_hbm.at[pl.ds(SHARED_DIM, GROWS), pl.ds(0, PIECE)],
                ring1_v, sem1).wait()
            process(ring1_v, p * PIECE, PIECE - 1)
        return carry

    lax.fori_loop(0, npieces, piece_step, 0)

    @pl.when(wid == 31)
    def _():
        pltpu.async_copy(
            table_hbm.at[pl.ds(SHARED_DIM, GROWS), pl.ds(TAIL512_LO, 512)],
            tail512_v, sem0).wait()
        process(tail512_v, 30 * PIECE, 511)
        pltpu.async_copy(
            table_hbm.at[pl.ds(SHARED_DIM, GROWS), pl.ds(TAIL64_LO, 64)],
            tail64_v, sem0).wait()
        process(tail64_v, 30 * PIECE + 512, 63)


@jax.jit
def kernel(X, table, shared_embed):
    idx = X.astype(jnp.int32)
    table_t = table.T
    mesh = plsc.VectorSubcoreMesh(core_axis_name="c", subcore_axis_name="s")
    k = pl.kernel(
        _body,
        mesh=mesh,
        compiler_params=pltpu.CompilerParams(
            use_tc_tiling_on_sc=True, needs_layout_passes=False
        ),
        out_type=jax.ShapeDtypeStruct((BATCH * EMBED_DIM,), jnp.float32),
        scratch_types=[
            pltpu.VMEM((BATCH,), jnp.int32),
            pltpu.VMEM((BATCH,), jnp.int32),
            pltpu.VMEM((GROWS, PIECE), jnp.float32),
            pltpu.VMEM((GROWS, PIECE), jnp.float32),
            pltpu.VMEM((GROWS, 512), jnp.float32),
            pltpu.VMEM((GROWS, 64), jnp.float32),
            pltpu.VMEM((L * EMBED_DIM,), jnp.float32),
            pltpu.VMEM((1, SHARED_DIM), jnp.float32),
            pltpu.SemaphoreType.DMA,
            pltpu.SemaphoreType.DMA,
            pltpu.SemaphoreType.DMA,
        ],
    )
    out_flat = k(idx, shared_embed, table_t)
    return out_flat.reshape(BATCH, EMBED_DIM)
